# Initial kernel scaffold; baseline (speedup 1.0000x reference)
import functools

import jax
import jax.numpy as jnp
from jax import lax
from jax.experimental import pallas as pl
from jax.experimental.pallas import tpu as pltpu
from jax.experimental.pallas import tpu_sc as plsc

NC = 2
NS = 16
LANES = 16


def _sc_mesh():
    return plsc.VectorSubcoreMesh(
        core_axis_name="c", subcore_axis_name="s", num_cores=NC, num_subcores=NS
    )


@functools.lru_cache(maxsize=None)
def _make_deg_kernel(n, e):
    nw = NC * NS
    assert e % nw == 0 and n % LANES == 0
    epw = e // nw
    blk = 4000

    @functools.partial(
        pl.kernel,
        out_type=jax.ShapeDtypeStruct((nw, n), jnp.float32),
        mesh=_sc_mesh(),
        scratch_types=[
            pltpu.VMEM((n,), jnp.float32),
            pltpu.VMEM((blk,), jnp.int32),
        ],
    )
    def deg_kernel(ei_hbm, out_hbm, hist, dbuf):
        s = lax.axis_index("s")
        c = lax.axis_index("c")
        w = s * NC + c
        zero = jnp.zeros((LANES,), jnp.float32)

        def zbody(i, carry):
            hist[pl.ds(i * LANES, LANES)] = zero
            return carry

        lax.fori_loop(0, n // LANES, zbody, 0)

        ones = jnp.ones((LANES,), jnp.float32)
        ebase = w * epw
        off = 0
        while off < epw:
            cblk = min(blk, epw - off)
            pltpu.sync_copy(
                ei_hbm.at[1, pl.ds(ebase + off, cblk)], dbuf.at[pl.ds(0, cblk)]
            )

            def body(i, carry):
                idx = dbuf[pl.ds(i * LANES, LANES)]
                plsc.addupdate_scatter(hist, [idx], ones)
                return carry

            lax.fori_loop(0, cblk // LANES, body, 0)
            off += cblk

        pltpu.sync_copy(hist, out_hbm.at[w])

    return deg_kernel


@functools.lru_cache(maxsize=None)
def _make_agg_kernel(n, e, f, c_rows, n_pass):
    assert e % NS == 0 and c_rows % (NS * LANES) == 0 and f % LANES == 0
    epw = e // NS
    blk = 4000
    assert epw % blk == 0
    cap = blk + 192
    nb = 128
    c_tot = c_rows + 64
    n_out = n_pass * NC * c_rows
    assert n_out >= n

    @functools.partial(
        pl.kernel,
        out_type=jax.ShapeDtypeStruct((n_out, f), jnp.float32),
        mesh=_sc_mesh(),
        scratch_types=[
            pltpu.VMEM_SHARED((c_tot, f), jnp.float32),
            pltpu.VMEM((blk,), jnp.int32),
            pltpu.VMEM((blk,), jnp.int32),
            pltpu.VMEM((cap,), jnp.int32),
            pltpu.VMEM((cap,), jnp.int32),
            pltpu.VMEM((nb,), jnp.int32),
            pltpu.VMEM((nb,), jnp.int32),
            pltpu.VMEM((nb, f), jnp.float32),
            pltpu.VMEM((64, f), jnp.float32),
            pltpu.SemaphoreType.DMA,
        ],
    )
    def agg_kernel(u_hbm, ei_hbm, out_hbm, acc, sbuf, dbuf, fsrc, fdst,
                   gstage, sstage, rows, zbuf, sem):
        s = lax.axis_index("s")
        c = lax.axis_index("c")
        w = s * NC + c
        zv = jnp.zeros((LANES,), jnp.float32)

        def zrow(i, carry):
            for j in range(f // LANES):
                zbuf[i, pl.ds(j * LANES, LANES)] = zv
            return carry

        lax.fori_loop(0, 64, zrow, 0)

        iota = lax.iota(jnp.int32, LANES)
        padsrc = (iota * 131 + w * 977) % n
        paddst = c_rows + iota

        rpt_tot = c_tot // NS
        rpt = c_rows // NS

        for p in range(n_pass):
            chunk = p * NC + c
            base = chunk * c_rows

            off = 0
            while off < rpt_tot:
                zb = min(64, rpt_tot - off)
                pltpu.sync_copy(
                    zbuf.at[pl.ds(0, zb)],
                    acc.at[pl.ds(s * rpt_tot + off, zb)],
                )
                off += zb
            plsc.subcore_barrier()

            ebase = s * epw
            cur = jnp.int32(0)
            for bi in range(epw // blk):
                off = ebase + bi * blk
                pltpu.sync_copy(ei_hbm.at[0, pl.ds(off, blk)], sbuf)
                pltpu.sync_copy(ei_hbm.at[1, pl.ds(off, blk)], dbuf)

                def cbody(i, cur):
                    srcv = sbuf[pl.ds(i * LANES, LANES)]
                    dstv = dbuf[pl.ds(i * LANES, LANES)]
                    m = (dstv >= base) & (dstv < base + c_rows)
                    plsc.store_compressed(fsrc.at[pl.ds(cur, LANES)], srcv,
                                          mask=m)
                    plsc.store_compressed(fdst.at[pl.ds(cur, LANES)],
                                          dstv - base, mask=m)
                    return cur + jnp.sum(m.astype(jnp.int32))

                cur = lax.fori_loop(0, blk // LANES, cbody, cur)

                nfull = cur // nb

                def fbody(k, carry):
                    kb = k * nb
                    for j in range(nb // LANES):
                        gstage[pl.ds(j * LANES, LANES)] = (
                            fsrc[pl.ds(kb + j * LANES, LANES)])
                        sstage[pl.ds(j * LANES, LANES)] = (
                            fdst[pl.ds(kb + j * LANES, LANES)])
                    pltpu.async_copy(u_hbm.at[gstage], rows, sem).wait()
                    pltpu.sync_copy(rows, acc.at[sstage], add=True)
                    return carry

                lax.fori_loop(0, nfull, fbody, 0)

                tb = nfull * nb
                for j in range(9):
                    fsrc[pl.ds(j * LANES, LANES)] = (
                        fsrc[pl.ds(tb + j * LANES, LANES)])
                    fdst[pl.ds(j * LANES, LANES)] = (
                        fdst[pl.ds(tb + j * LANES, LANES)])
                cur = cur - tb

            @pl.when(cur > 0)
            def _final():
                for j in range(nb // LANES):
                    fsrc[pl.ds(cur + j * LANES, LANES)] = padsrc
                    fdst[pl.ds(cur + j * LANES, LANES)] = paddst
                for j in range(nb // LANES):
                    gstage[pl.ds(j * LANES, LANES)] = (
                        fsrc[pl.ds(j * LANES, LANES)])
                    sstage[pl.ds(j * LANES, LANES)] = (
                        fdst[pl.ds(j * LANES, LANES)])
                pltpu.async_copy(u_hbm.at[gstage], rows, sem).wait()
                pltpu.sync_copy(rows, acc.at[sstage], add=True)

            plsc.subcore_barrier()

            obase = chunk * c_rows
            off = 0
            while off < rpt:
                fb = min(nb, rpt - off)
                pltpu.sync_copy(acc.at[pl.ds(s * rpt + off, fb)],
                                rows.at[pl.ds(0, fb)])
                pltpu.sync_copy(rows.at[pl.ds(0, fb)],
                                out_hbm.at[pl.ds(obase + s * rpt + off, fb)])
                off += fb
            plsc.subcore_barrier()

    return agg_kernel


def kernel(x, edge_index, edge_attr, batch, W1, b1, W2, b2, W3, b3,
           Wf1, bf1, Wf2, bf2, gamma, beta):
    n = x.shape[0]
    e = edge_index.shape[1]
    num_graphs = 256

    deg_parts = _make_deg_kernel(n, e)(edge_index)
    deg = deg_parts.sum(0) + 1.0
    dinv = jax.lax.rsqrt(deg)

    agg64 = _make_agg_kernel(n, e, 64, 29696, 2)
    agg128 = _make_agg_kernel(n, e, 128, 14336, 4)

    u1 = (x @ W1) * dinv[:, None]
    agg1 = agg64(u1, edge_index)[:n]
    h1 = jax.nn.relu((agg1 + u1) * dinv[:, None] + b1)

    u2 = (h1 @ W2) * dinv[:, None]
    agg2 = agg128(u2, edge_index)[:n]
    h2 = jax.nn.relu((agg2 + u2) * dinv[:, None] + b2)

    u3 = (h2 @ W3) * dinv[:, None]
    agg3 = agg64(u3, edge_index)[:n]
    h3 = jax.nn.relu((agg3 + u3) * dinv[:, None] + b3)

    g_num = jax.ops.segment_sum(h3, batch, num_segments=num_graphs)
    cnt = jax.ops.segment_sum(jnp.ones((n,), jnp.float32), batch,
                              num_segments=num_graphs)
    g = g_num / jnp.clip(cnt, 1.0)[:, None]
    g = jax.nn.relu(g @ Wf1 + bf1)
    g = jax.nn.relu(g @ Wf2 + bf2)
    mu = jnp.mean(g, axis=-1, keepdims=True)
    var = jnp.mean((g - mu) ** 2, axis=-1, keepdims=True)
    return (g - mu) / jnp.sqrt(var + 1e-5) * gamma + beta


# speedup vs baseline: 12.2174x; 12.2174x over previous
import functools

import jax
import jax.numpy as jnp
from jax import lax
from jax.experimental import pallas as pl
from jax.experimental.pallas import tpu as pltpu
from jax.experimental.pallas import tpu_sc as plsc

NC = 2
NS = 16
LANES = 16


def _sc_mesh():
    return plsc.VectorSubcoreMesh(
        core_axis_name="c", subcore_axis_name="s", num_cores=NC, num_subcores=NS
    )


@functools.lru_cache(maxsize=None)
def _make_deg_kernel(n, e):
    nw = NC * NS
    assert e % nw == 0 and n % LANES == 0
    epw = e // nw
    blk = 2000
    assert epw % blk == 0 and blk % LANES == 0

    @functools.partial(
        pl.kernel,
        out_type=jax.ShapeDtypeStruct((nw, n), jnp.float32),
        mesh=_sc_mesh(),
        scratch_types=[
            pltpu.VMEM((n,), jnp.float32),
            pltpu.VMEM((blk,), jnp.int32),
        ],
        compiler_params=pltpu.CompilerParams(needs_layout_passes=False, use_tc_tiling_on_sc=False),
    )
    def deg_kernel(dst_hbm, out_hbm, hist, dbuf):
        s = lax.axis_index("s")
        c = lax.axis_index("c")
        w = s * NC + c
        zero = jnp.zeros((LANES,), jnp.float32)

        def zbody(i, carry):
            hist[pl.ds(i * LANES, LANES)] = zero
            return carry

        lax.fori_loop(0, n // LANES, zbody, 0)

        ones = jnp.ones((LANES,), jnp.float32)
        ebase = w * epw

        def blk_body(bi, carry):
            pltpu.sync_copy(
                dst_hbm.at[pl.ds(ebase + bi * blk, blk)], dbuf
            )

            def body(i, c2):
                idx = dbuf[pl.ds(i * LANES, LANES)]
                plsc.addupdate_scatter(hist, [idx], ones)
                return c2

            lax.fori_loop(0, blk // LANES, body, 0)
            return carry

        lax.fori_loop(0, epw // blk, blk_body, 0)

        pltpu.sync_copy(hist, out_hbm.at[w])

    return deg_kernel


@functools.lru_cache(maxsize=None)
def _make_agg_kernel(n, e, f, c_rows, n_pass):
    assert e % NS == 0 and c_rows % (NS * LANES) == 0 and f % LANES == 0
    epw = e // NS
    blk = 2000
    assert epw % blk == 0
    cap = blk + 192
    nb = 128 if f <= 64 else 64
    c_tot = c_rows + 64
    n_out = n_pass * NC * c_rows
    assert n_out >= n

    @functools.partial(
        pl.kernel,
        out_type=jax.ShapeDtypeStruct((n_out, f), jnp.float32),
        mesh=_sc_mesh(),
        scratch_types=[
            pltpu.VMEM_SHARED((c_tot, f), jnp.float32),
            pltpu.VMEM((blk,), jnp.int32),
            pltpu.VMEM((blk,), jnp.int32),
            pltpu.VMEM((cap,), jnp.int32),
            pltpu.VMEM((cap,), jnp.int32),
            pltpu.VMEM((nb,), jnp.int32),
            pltpu.VMEM((nb,), jnp.int32),
            pltpu.VMEM((nb, f), jnp.float32),
            pltpu.VMEM((32, f), jnp.float32),
            pltpu.SemaphoreType.DMA,
        ],
        compiler_params=pltpu.CompilerParams(needs_layout_passes=False, use_tc_tiling_on_sc=False),
    )
    def agg_kernel(u_hbm, src_hbm, dst_hbm, out_hbm, acc, sbuf, dbuf, fsrc, fdst,
                   gstage, sstage, rows, zbuf, sem):
        s = lax.axis_index("s")
        c = lax.axis_index("c")
        w = s * NC + c
        zv = jnp.zeros((LANES,), jnp.float32)

        def zrow(i, carry):
            for j in range(f // LANES):
                zbuf[i, pl.ds(j * LANES, LANES)] = zv
            return carry

        lax.fori_loop(0, 32, zrow, 0)

        iota = lax.iota(jnp.int32, LANES)
        padsrc = (iota * 131 + w * 977) % n
        paddst = c_rows + iota

        rpt_tot = c_tot // NS
        rpt = c_rows // NS

        def pass_body(p, carry):
            chunk = p * NC + c
            base = chunk * c_rows

            nz = rpt_tot // 32

            def zb_body(i, c2):
                pltpu.sync_copy(zbuf, acc.at[pl.ds(s * rpt_tot + i * 32, 32)])
                return c2

            lax.fori_loop(0, nz, zb_body, 0)
            ztail = rpt_tot - nz * 32
            if ztail:
                pltpu.sync_copy(zbuf.at[pl.ds(0, ztail)],
                                acc.at[pl.ds(s * rpt_tot + nz * 32, ztail)])
            plsc.subcore_barrier()

            ebase = s * epw

            def blk_body(bi, cur):
                off = ebase + bi * blk
                pltpu.sync_copy(src_hbm.at[pl.ds(off, blk)], sbuf)
                pltpu.sync_copy(dst_hbm.at[pl.ds(off, blk)], dbuf)

                def cbody(i, cur):
                    srcv = sbuf[pl.ds(i * LANES, LANES)]
                    dstv = dbuf[pl.ds(i * LANES, LANES)]
                    m = (dstv >= base) & (dstv < base + c_rows)
                    plsc.store_compressed(fsrc.at[pl.ds(cur, LANES)], srcv,
                                          mask=m)
                    plsc.store_compressed(fdst.at[pl.ds(cur, LANES)],
                                          dstv - base, mask=m)
                    return cur + jnp.sum(m.astype(jnp.int32))

                cur = lax.fori_loop(0, blk // LANES, cbody, cur)

                nfull = cur // nb

                def fbody(k, c2):
                    kb = k * nb
                    for j in range(nb // LANES):
                        gstage[pl.ds(j * LANES, LANES)] = (
                            fsrc[pl.ds(kb + j * LANES, LANES)])
                        sstage[pl.ds(j * LANES, LANES)] = (
                            fdst[pl.ds(kb + j * LANES, LANES)])
                    pltpu.async_copy(u_hbm.at[gstage], rows, sem).wait()
                    pltpu.sync_copy(rows, acc.at[sstage], add=True)
                    return c2

                lax.fori_loop(0, nfull, fbody, 0)

                tb = nfull * nb
                for j in range(9):
                    fsrc[pl.ds(j * LANES, LANES)] = (
                        fsrc[pl.ds(tb + j * LANES, LANES)])
                    fdst[pl.ds(j * LANES, LANES)] = (
                        fdst[pl.ds(tb + j * LANES, LANES)])
                return cur - tb

            cur = lax.fori_loop(0, epw // blk, blk_body, jnp.int32(0))

            @pl.when(cur > 0)
            def _final():
                for j in range(nb // LANES):
                    fsrc[pl.ds(cur + j * LANES, LANES)] = padsrc
                    fdst[pl.ds(cur + j * LANES, LANES)] = paddst
                for j in range(nb // LANES):
                    gstage[pl.ds(j * LANES, LANES)] = (
                        fsrc[pl.ds(j * LANES, LANES)])
                    sstage[pl.ds(j * LANES, LANES)] = (
                        fdst[pl.ds(j * LANES, LANES)])
                pltpu.async_copy(u_hbm.at[gstage], rows, sem).wait()
                pltpu.sync_copy(rows, acc.at[sstage], add=True)

            plsc.subcore_barrier()

            obase = chunk * c_rows
            nf = rpt // nb

            def fl_body(i, c2):
                r = s * rpt + i * nb
                pltpu.sync_copy(acc.at[pl.ds(r, nb)], rows)
                pltpu.sync_copy(rows, out_hbm.at[pl.ds(obase + r, nb)])
                return c2

            lax.fori_loop(0, nf, fl_body, 0)
            ftail = rpt - nf * nb
            if ftail:
                r = s * rpt + nf * nb
                pltpu.sync_copy(acc.at[pl.ds(r, ftail)],
                                rows.at[pl.ds(0, ftail)])
                pltpu.sync_copy(rows.at[pl.ds(0, ftail)],
                                out_hbm.at[pl.ds(obase + r, ftail)])
            plsc.subcore_barrier()
            return carry

        lax.fori_loop(0, n_pass, pass_body, 0)

    return agg_kernel


def kernel(x, edge_index, edge_attr, batch, W1, b1, W2, b2, W3, b3,
           Wf1, bf1, Wf2, bf2, gamma, beta):
    n = x.shape[0]
    e = edge_index.shape[1]
    num_graphs = 256

    e_src = edge_index[0]
    e_dst = edge_index[1]
    deg_parts = _make_deg_kernel(n, e)(e_dst)
    deg = deg_parts.sum(0) + 1.0
    dinv = jax.lax.rsqrt(deg)

    agg64 = _make_agg_kernel(n, e, 64, 25088, 2)
    agg128 = _make_agg_kernel(n, e, 128, 13568, 4)

    u1 = (x @ W1) * dinv[:, None]
    agg1 = agg64(u1, e_src, e_dst)[:n]
    h1 = jax.nn.relu((agg1 + u1) * dinv[:, None] + b1)

    u2 = (h1 @ W2) * dinv[:, None]
    agg2 = agg128(u2, e_src, e_dst)[:n]
    h2 = jax.nn.relu((agg2 + u2) * dinv[:, None] + b2)

    u3 = (h2 @ W3) * dinv[:, None]
    agg3 = agg64(u3, e_src, e_dst)[:n]
    h3 = jax.nn.relu((agg3 + u3) * dinv[:, None] + b3)

    g_num = jax.ops.segment_sum(h3, batch, num_segments=num_graphs)
    cnt = jax.ops.segment_sum(jnp.ones((n,), jnp.float32), batch,
                              num_segments=num_graphs)
    g = g_num / jnp.clip(cnt, 1.0)[:, None]
    g = jax.nn.relu(g @ Wf1 + bf1)
    g = jax.nn.relu(g @ Wf2 + bf2)
    mu = jnp.mean(g, axis=-1, keepdims=True)
    var = jnp.mean((g - mu) ** 2, axis=-1, keepdims=True)
    return (g - mu) / jnp.sqrt(var + 1e-5) * gamma + beta
